# Initial kernel scaffold; baseline (speedup 1.0000x reference)
import functools

import jax
import jax.numpy as jnp
from jax import lax
from jax.experimental import pallas as pl
from jax.experimental.pallas import tpu as pltpu
from jax.experimental.pallas import tpu_sc as plsc

_VOCAB = 1000000
_EMBED_DIM = 32
_BATCH = 16384
_HIST = 50
_B = _BATCH * _HIST

_info = plsc.get_sparse_core_info()
_NC, _NS = _info.num_cores, _info.num_subcores
_NW = _NC * _NS
_B_PER_W = _B // _NW
_CH = 1024
_N_CHUNKS = _B_PER_W // _CH


def _make_kernel():
    mesh = plsc.VectorSubcoreMesh(core_axis_name="c", subcore_axis_name="s")

    @functools.partial(
        pl.kernel,
        out_type=jax.ShapeDtypeStruct((_B, _EMBED_DIM), jnp.float32),
        mesh=mesh,
        scratch_types=[
            pltpu.VMEM((_CH,), jnp.int32),
            pltpu.VMEM((_CH, _EMBED_DIM), jnp.float32),
            pltpu.SemaphoreType.DMA,
        ],
    )
    def gather_kernel(table_hbm, idx_hbm, out_hbm, idx_v, rows_v, sem):
        wid = lax.axis_index("s") * _NC + lax.axis_index("c")
        w_base = wid * _B_PER_W

        def chunk_body(c, carry):
            base = w_base + c * _CH
            pltpu.sync_copy(idx_hbm.at[pl.ds(base, _CH)], idx_v)
            pltpu.async_copy(table_hbm.at[idx_v], rows_v, sem).wait()
            pltpu.sync_copy(rows_v, out_hbm.at[pl.ds(base, _CH)])
            return carry

        lax.fori_loop(0, _N_CHUNKS, chunk_body, 0)

    return gather_kernel


_gather = _make_kernel()


def kernel(indices, weight):
    flat_idx = indices.reshape(_B).astype(jnp.int32)
    out = _gather(weight, flat_idx)
    return out.reshape(_BATCH, _HIST, _EMBED_DIM)


# speedup vs baseline: 1.0936x; 1.0936x over previous
import functools

import jax
import jax.numpy as jnp
from jax import lax
from jax.experimental import pallas as pl
from jax.experimental.pallas import tpu as pltpu
from jax.experimental.pallas import tpu_sc as plsc

_VOCAB = 1000000
_EMBED_DIM = 32
_BATCH = 16384
_HIST = 50
_B = _BATCH * _HIST

_info = plsc.get_sparse_core_info()
_NC, _NS = _info.num_cores, _info.num_subcores
_NW = _NC * _NS
_B_PER_W = _B // _NW
_CH = 1024
_N_CHUNKS = _B_PER_W // _CH


def _make_kernel():
    mesh = plsc.VectorSubcoreMesh(core_axis_name="c", subcore_axis_name="s")

    @functools.partial(
        pl.kernel,
        out_type=jax.ShapeDtypeStruct((_B, _EMBED_DIM), jnp.float32),
        mesh=mesh,
        scratch_types=[
            pltpu.VMEM((_CH,), jnp.int32),
            pltpu.VMEM((_CH, _EMBED_DIM), jnp.float32),
            pltpu.SemaphoreType.DMA,
        ],
        compiler_params=pltpu.CompilerParams(use_tc_tiling_on_sc=False),
    )
    def gather_kernel(table_hbm, idx_hbm, out_hbm, idx_v, rows_v, sem):
        wid = lax.axis_index("s") * _NC + lax.axis_index("c")
        w_base = wid * _B_PER_W

        def chunk_body(c, carry):
            base = w_base + c * _CH
            pltpu.sync_copy(idx_hbm.at[pl.ds(base, _CH)], idx_v)
            pltpu.async_copy(table_hbm.at[idx_v], rows_v, sem).wait()
            pltpu.sync_copy(rows_v, out_hbm.at[pl.ds(base, _CH)])
            return carry

        lax.fori_loop(0, _N_CHUNKS, chunk_body, 0)

    return gather_kernel


_gather = _make_kernel()


def kernel(indices, weight):
    flat_idx = indices.reshape(_B).astype(jnp.int32)
    out = _gather(weight, flat_idx)
    return out.reshape(_BATCH, _HIST, _EMBED_DIM)


# speedup vs baseline: 1.1077x; 1.0129x over previous
import functools

import jax
import jax.numpy as jnp
from jax import lax
from jax.experimental import pallas as pl
from jax.experimental.pallas import tpu as pltpu
from jax.experimental.pallas import tpu_sc as plsc

_VOCAB = 1000000
_EMBED_DIM = 32
_BATCH = 16384
_HIST = 50
_B = _BATCH * _HIST

_info = plsc.get_sparse_core_info()
_NC, _NS = _info.num_cores, _info.num_subcores
_NW = _NC * _NS
_B_PER_W = _B // _NW
_CH = 1600
_NB = 2
_N_CHUNKS = _B_PER_W // _CH
_N_OUTER = _N_CHUNKS // _NB
assert _N_CHUNKS * _CH == _B_PER_W and _N_OUTER * _NB == _N_CHUNKS


def _make_kernel():
    mesh = plsc.VectorSubcoreMesh(core_axis_name="c", subcore_axis_name="s")

    @functools.partial(
        pl.kernel,
        out_type=jax.ShapeDtypeStruct((_B, _EMBED_DIM), jnp.float32),
        mesh=mesh,
        scratch_types=(
            [pltpu.VMEM((_CH,), jnp.int32)]
            + [pltpu.VMEM((_CH, _EMBED_DIM), jnp.float32) for _ in range(_NB)]
            + [pltpu.SemaphoreType.DMA, pltpu.SemaphoreType.DMA, pltpu.SemaphoreType.DMA]
        ),
        compiler_params=pltpu.CompilerParams(use_tc_tiling_on_sc=False),
    )
    def gather_kernel(table_hbm, idx_hbm, out_hbm, idx_v, rows0, rows1, gsem, osem0, osem1):
        row_bufs = [rows0, rows1]
        osems = [osem0, osem1]
        wid = lax.axis_index("s") * _NC + lax.axis_index("c")
        w_base = wid * _B_PER_W

        def out_copy(g, b):
            return pltpu.make_async_copy(
                row_bufs[b], out_hbm.at[pl.ds(w_base + g * _CH, _CH)], osems[b]
            )

        def load_and_gather(g, b):
            pltpu.sync_copy(idx_hbm.at[pl.ds(w_base + g * _CH, _CH)], idx_v)
            gather = pltpu.make_async_copy(table_hbm.at[idx_v], row_bufs[b], gsem)
            gather.start()
            gather.wait()

        for b in range(_NB):
            load_and_gather(b, b)
            out_copy(b, b).start()

        def outer(g0, carry):
            for b in range(_NB):
                g = g0 * _NB + b
                out_copy(g - _NB, b).wait()
                load_and_gather(g, b)
                out_copy(g, b).start()
            return carry

        lax.fori_loop(1, _N_OUTER, outer, 0)
        for b in range(_NB):
            out_copy((_N_OUTER - 1) * _NB + b, b).wait()

    return gather_kernel


_gather = _make_kernel()


def kernel(indices, weight):
    flat_idx = indices.reshape(_B).astype(jnp.int32)
    out = _gather(weight, flat_idx)
    return out.reshape(_BATCH, _HIST, _EMBED_DIM)


# speedup vs baseline: 1.1118x; 1.0036x over previous
import functools

import jax
import jax.numpy as jnp
from jax import lax
from jax.experimental import pallas as pl
from jax.experimental.pallas import tpu as pltpu
from jax.experimental.pallas import tpu_sc as plsc

_VOCAB = 1000000
_EMBED_DIM = 32
_BATCH = 16384
_HIST = 50
_B = _BATCH * _HIST

_info = plsc.get_sparse_core_info()
_NC, _NS_SUB = _info.num_cores, _info.num_subcores
_NW = _NC * _NS_SUB
_B_PER_W = _B // _NW
_NS = 4
_SUB = 800
_N_CHUNKS = _B_PER_W // _SUB
_N_ROUNDS = _N_CHUNKS // _NS
assert _NS * _SUB * _N_ROUNDS == _B_PER_W
assert (_B - _SUB) % 8 == 0 and _SUB % 8 == 0


def _make_kernel():
    mesh = plsc.VectorSubcoreMesh(core_axis_name="c", subcore_axis_name="s")

    @functools.partial(
        pl.kernel,
        out_type=jax.ShapeDtypeStruct((_B, _EMBED_DIM), jnp.float32),
        mesh=mesh,
        scratch_types=(
            [pltpu.VMEM((_SUB,), jnp.int32) for _ in range(_NS)]
            + [pltpu.VMEM((_SUB, _EMBED_DIM), jnp.float32) for _ in range(_NS)]
            + [pltpu.SemaphoreType.DMA for _ in range(3 * _NS)]
        ),
        compiler_params=pltpu.CompilerParams(use_tc_tiling_on_sc=False),
    )
    def gather_kernel(table_hbm, idx_hbm, out_hbm, *scratch):
        idx_bufs = scratch[:_NS]
        row_bufs = scratch[_NS : 2 * _NS]
        isems = scratch[2 * _NS : 3 * _NS]
        gsems = scratch[3 * _NS : 4 * _NS]
        osems = scratch[4 * _NS : 5 * _NS]
        wid = lax.axis_index("s") * _NC + lax.axis_index("c")
        w_base = wid * _B_PER_W

        def idx_copy(g, s):
            off = jnp.minimum(w_base + g * _SUB, _B - _SUB)
            return pltpu.make_async_copy(
                idx_hbm.at[pl.ds(off, _SUB)], idx_bufs[s], isems[s]
            )

        def gather(s):
            return pltpu.make_async_copy(
                table_hbm.at[idx_bufs[s]], row_bufs[s], gsems[s]
            )

        def out_copy(g, s):
            return pltpu.make_async_copy(
                row_bufs[s], out_hbm.at[pl.ds(w_base + g * _SUB, _SUB)], osems[s]
            )

        for s in range(_NS):
            idx_copy(s, s).start()
        for s in range(_NS):
            idx_copy(s, s).wait()
            gather(s).start()
        for s in range(_NS):
            gather(s).wait()
            out_copy(s, s).start()
            idx_copy(_NS + s, s).start()

        def round_body(r, carry):
            for s in range(_NS):
                g = r * _NS + s
                out_copy(g - _NS, s).wait()
                idx_copy(g, s).wait()
                gather(s).start()
            for s in range(_NS):
                g = r * _NS + s
                gather(s).wait()
                out_copy(g, s).start()
                idx_copy(g + _NS, s).start()
            return carry

        lax.fori_loop(1, _N_ROUNDS, round_body, 0)

        for s in range(_NS):
            out_copy((_N_ROUNDS - 1) * _NS + s, s).wait()
            idx_copy(0, s).wait()

    return gather_kernel


_gather = _make_kernel()


def kernel(indices, weight):
    flat_idx = indices.reshape(_B).astype(jnp.int32)
    out = _gather(weight, flat_idx)
    return out.reshape(_BATCH, _HIST, _EMBED_DIM)


# speedup vs baseline: 1.7953x; 1.6148x over previous
import functools

import jax
import jax.numpy as jnp
from jax import lax
from jax.experimental import pallas as pl
from jax.experimental.pallas import tpu as pltpu
from jax.experimental.pallas import tpu_sc as plsc

_VOCAB = 1000000
_EMBED_DIM = 32
_BATCH = 16384
_HIST = 50

_info = plsc.get_sparse_core_info()
_NC, _NS_SUB = _info.num_cores, _info.num_subcores
_NW = _NC * _NS_SUB
_BPW = _BATCH // _NW
_NBS = 16
_NSLOT = 2
_N_ROUNDS = _BPW // (_NBS * _NSLOT)
assert _NBS * _NSLOT * _N_ROUNDS == _BPW


def _make_kernel():
    mesh = plsc.VectorSubcoreMesh(core_axis_name="c", subcore_axis_name="s")

    @functools.partial(
        pl.kernel,
        out_type=jax.ShapeDtypeStruct((_BATCH, _HIST, _EMBED_DIM), jnp.float32),
        mesh=mesh,
        scratch_types=(
            [pltpu.VMEM((_NBS, _HIST), jnp.int32) for _ in range(_NSLOT)]
            + [pltpu.VMEM((_NBS, _HIST, _EMBED_DIM), jnp.float32) for _ in range(_NSLOT)]
            + [pltpu.SemaphoreType.DMA for _ in range(3 * _NSLOT)]
        ),
        compiler_params=pltpu.CompilerParams(use_tc_tiling_on_sc=False),
    )
    def gather_kernel(table_hbm, idx_hbm, out_hbm, *scratch):
        idx_bufs = scratch[:_NSLOT]
        row_bufs = scratch[_NSLOT : 2 * _NSLOT]
        isems = scratch[2 * _NSLOT : 3 * _NSLOT]
        gsems = scratch[3 * _NSLOT : 4 * _NSLOT]
        osems = scratch[4 * _NSLOT : 5 * _NSLOT]
        wid = lax.axis_index("s") * _NC + lax.axis_index("c")
        w_base = wid * _BPW

        def idx_copy(g, s):
            off = jnp.minimum(w_base + g * _NBS, _BATCH - _NBS)
            return pltpu.make_async_copy(
                idx_hbm.at[pl.ds(off, _NBS)], idx_bufs[s], isems[s]
            )

        def gathers(s):
            return [
                pltpu.make_async_copy(
                    table_hbm.at[idx_bufs[s].at[b]], row_bufs[s].at[b], gsems[s]
                )
                for b in range(_NBS)
            ]

        def out_copy(g, s):
            return pltpu.make_async_copy(
                row_bufs[s],
                out_hbm.at[pl.ds(w_base + g * _NBS, _NBS)],
                osems[s],
            )

        for s in range(_NSLOT):
            idx_copy(s, s).start()
        for s in range(_NSLOT):
            idx_copy(s, s).wait()
            for gth in gathers(s):
                gth.start()
        for s in range(_NSLOT):
            for gth in gathers(s):
                gth.wait()
            out_copy(s, s).start()
            idx_copy(_NSLOT + s, s).start()

        def round_body(r, carry):
            for s in range(_NSLOT):
                g = r * _NSLOT + s
                out_copy(g - _NSLOT, s).wait()
                idx_copy(g, s).wait()
                for gth in gathers(s):
                    gth.start()
            for s in range(_NSLOT):
                g = r * _NSLOT + s
                for gth in gathers(s):
                    gth.wait()
                out_copy(g, s).start()
                idx_copy(g + _NSLOT, s).start()
            return carry

        lax.fori_loop(1, _N_ROUNDS, round_body, 0)

        for s in range(_NSLOT):
            out_copy((_N_ROUNDS - 1) * _NSLOT + s, s).wait()
            idx_copy(0, s).wait()

    return gather_kernel


_gather = _make_kernel()


def kernel(indices, weight):
    return _gather(weight, indices.astype(jnp.int32))


# speedup vs baseline: 1.8002x; 1.0027x over previous
import functools

import jax
import jax.numpy as jnp
from jax import lax
from jax.experimental import pallas as pl
from jax.experimental.pallas import tpu as pltpu
from jax.experimental.pallas import tpu_sc as plsc

_VOCAB = 1000000
_EMBED_DIM = 32
_BATCH = 16384
_HIST = 50

_info = plsc.get_sparse_core_info()
_NC, _NS_SUB = _info.num_cores, _info.num_subcores
_NW = _NC * _NS_SUB
_BPW = _BATCH // _NW
_NBS = 32
_NSLOT = 2
_N_ROUNDS = _BPW // (_NBS * _NSLOT)
assert _NBS * _NSLOT * _N_ROUNDS == _BPW


def _make_kernel():
    mesh = plsc.VectorSubcoreMesh(core_axis_name="c", subcore_axis_name="s")

    @functools.partial(
        pl.kernel,
        out_type=jax.ShapeDtypeStruct((_BATCH, _HIST, _EMBED_DIM), jnp.float32),
        mesh=mesh,
        scratch_types=(
            [pltpu.VMEM((_NBS, _HIST), jnp.int32) for _ in range(_NSLOT)]
            + [pltpu.VMEM((_NBS, _HIST, _EMBED_DIM), jnp.float32) for _ in range(_NSLOT)]
            + [pltpu.SemaphoreType.DMA for _ in range(3 * _NSLOT)]
        ),
        compiler_params=pltpu.CompilerParams(use_tc_tiling_on_sc=False),
    )
    def gather_kernel(table_hbm, idx_hbm, out_hbm, *scratch):
        idx_bufs = scratch[:_NSLOT]
        row_bufs = scratch[_NSLOT : 2 * _NSLOT]
        isems = scratch[2 * _NSLOT : 3 * _NSLOT]
        gsems = scratch[3 * _NSLOT : 4 * _NSLOT]
        osems = scratch[4 * _NSLOT : 5 * _NSLOT]
        wid = lax.axis_index("s") * _NC + lax.axis_index("c")
        w_base = wid * _BPW

        def idx_copy(g, s):
            off = jnp.minimum(w_base + g * _NBS, _BATCH - _NBS)
            return pltpu.make_async_copy(
                idx_hbm.at[pl.ds(off, _NBS)], idx_bufs[s], isems[s]
            )

        def gathers(s):
            return [
                pltpu.make_async_copy(
                    table_hbm.at[idx_bufs[s].at[b]], row_bufs[s].at[b], gsems[s]
                )
                for b in range(_NBS)
            ]

        def out_copy(g, s):
            return pltpu.make_async_copy(
                row_bufs[s],
                out_hbm.at[pl.ds(w_base + g * _NBS, _NBS)],
                osems[s],
            )

        for s in range(_NSLOT):
            idx_copy(s, s).start()
        for s in range(_NSLOT):
            idx_copy(s, s).wait()
            for gth in gathers(s):
                gth.start()
        for s in range(_NSLOT):
            for gth in gathers(s):
                gth.wait()
            out_copy(s, s).start()
            idx_copy(_NSLOT + s, s).start()

        def round_body(r, carry):
            for s in range(_NSLOT):
                g = r * _NSLOT + s
                out_copy(g - _NSLOT, s).wait()
                idx_copy(g, s).wait()
                for gth in gathers(s):
                    gth.start()
            for s in range(_NSLOT):
                g = r * _NSLOT + s
                for gth in gathers(s):
                    gth.wait()
                out_copy(g, s).start()
                idx_copy(g + _NSLOT, s).start()
            return carry

        lax.fori_loop(1, _N_ROUNDS, round_body, 0)

        for s in range(_NSLOT):
            out_copy((_N_ROUNDS - 1) * _NSLOT + s, s).wait()
            idx_copy(0, s).wait()

    return gather_kernel


_gather = _make_kernel()


def kernel(indices, weight):
    return _gather(weight, indices.astype(jnp.int32))
